# baseline (speedup 1.0000x reference)
import jax
import jax.numpy as jnp
from jax.experimental import pallas as pl


def _copy_body(x_ref, o_ref):
    o_ref[...] = x_ref[...]


def kernel(features, is_start):
    del is_start
    shape = features.shape
    total = features.size
    lanes = 128
    rows = total // lanes
    assert rows * lanes == total
    grid = 8
    block_rows = rows // grid
    x2d = features.reshape(rows, lanes)
    out = pl.pallas_call(
        _copy_body,
        grid=(grid,),
        in_specs=[pl.BlockSpec((block_rows, lanes), lambda i: (i, 0))],
        out_specs=pl.BlockSpec((block_rows, lanes), lambda i: (i, 0)),
        out_shape=jax.ShapeDtypeStruct((rows, lanes), features.dtype),
    )(x2d)
    return out.reshape(shape)
